# baseline (speedup 1.0000x reference)
import functools

import jax
import jax.numpy as jnp
from jax import lax
from jax.experimental import pallas as pl
from jax.experimental.pallas import tpu as pltpu
from jax.experimental.pallas import tpu_sc as plsc

N_CLS = 1000
N_CTX = 4
CTX_DIM = 512
CTX_LEN = 77
SUFFIX_LEN = CTX_LEN - 1 - N_CTX
_LANES = 16
_COLS = CTX_DIM // _LANES

_NC = 2
_NS = 16
_NW = _NC * _NS
_BASE = N_CLS // _NW
_REM = N_CLS % _NW
_NBUF = 3
_TRIPS = (_BASE + 1 + _NBUF - 1) // _NBUF


def _body(prefix_hbm, ctx_hbm, suffix_hbm, out_hbm,
          buf0, buf1, buf2, pbuf0, pbuf1, pbuf2, cbuf,
          semg0, semg1, semg2, semp0, semp1, semp2, semo0, semo1, semo2):
    bufs = (buf0, buf1, buf2)
    pbufs = (pbuf0, pbuf1, pbuf2)
    semg = (semg0, semg1, semg2)
    semp = (semp0, semp1, semp2)
    semo = (semo0, semo1, semo2)

    wid = lax.axis_index("s") * _NC + lax.axis_index("c")
    lo = wid * _BASE + jnp.minimum(wid, _REM)
    cnt = _BASE + jnp.where(wid < _REM, 1, 0)

    pltpu.sync_copy(ctx_hbm, cbuf)

    def issue(b, i):
        pltpu.async_copy(suffix_hbm.at[i], bufs[b].at[pl.ds(0, SUFFIX_LEN)], semg[b])
        pltpu.async_copy(prefix_hbm.at[i], pbufs[b], semp[b])

    def wait_gather(b):
        pltpu.make_async_copy(suffix_hbm.at[lo], bufs[b].at[pl.ds(0, SUFFIX_LEN)], semg[b]).wait()
        pltpu.make_async_copy(prefix_hbm.at[lo], pbufs[b], semp[b]).wait()

    def wait_scatter(b):
        pltpu.make_async_copy(bufs[b], out_hbm.at[lo], semo[b]).wait()

    for b in range(_NBUF):
        issue(b, lo + b)

    def copy_rows(dst_ref, dst_rows, src_ref, src_rows):
        vals = [(dr, [src_ref[sr, pl.ds(c * _LANES, _LANES)] for c in range(_COLS)])
                for dr, sr in zip(dst_rows, src_rows)]
        for dr, vs in vals:
            for c in range(_COLS):
                dst_ref[dr, pl.ds(c * _LANES, _LANES)] = vs[c]

    def turn(k, carry):
        for s in range(_NBUF):
            j = k * _NBUF + s
            i = lo + j

            br = (s + 2) % _NBUF
            rf = j + 2

            def refill():
                wait_scatter(br)
                issue(br, lo + rf)

            jax.lax.cond(jnp.logical_and(rf >= _NBUF, rf < cnt), refill,
                         lambda: None)

            def slot():
                wait_gather(s)

                def shift_rows(t, c2):
                    r = (SUFFIX_LEN - 1) - 2 * t
                    copy_rows(bufs[s],
                              (r + (N_CTX + 1), r + N_CTX),
                              bufs[s],
                              (r, r - 1))
                    return c2

                lax.fori_loop(0, SUFFIX_LEN // 2, shift_rows, 0)

                copy_rows(bufs[s], (0,), pbufs[s], (0,))
                copy_rows(bufs[s], (1, 2), cbuf, (0, 1))
                copy_rows(bufs[s], (3, 4), cbuf, (2, 3))

                pltpu.async_copy(bufs[s], out_hbm.at[i], semo[s])


            if s == 0:
                slot()
            else:
                jax.lax.cond(j < cnt, slot, lambda: None)
        return carry

    lax.fori_loop(0, _TRIPS, turn, 0)

    for b in range(_NBUF):
        wait_scatter(b)


def kernel(prefixs, ctx, suffixs):
    mesh = plsc.VectorSubcoreMesh(core_axis_name="c", subcore_axis_name="s")
    run = pl.kernel(
        _body,
        out_type=jax.ShapeDtypeStruct((N_CLS, CTX_LEN, CTX_DIM), jnp.float32),
        mesh=mesh,
        scratch_types=[
            pltpu.VMEM((CTX_LEN, CTX_DIM), jnp.float32),
            pltpu.VMEM((CTX_LEN, CTX_DIM), jnp.float32),
            pltpu.VMEM((CTX_LEN, CTX_DIM), jnp.float32),
            pltpu.VMEM((1, CTX_DIM), jnp.float32),
            pltpu.VMEM((1, CTX_DIM), jnp.float32),
            pltpu.VMEM((1, CTX_DIM), jnp.float32),
            pltpu.VMEM((N_CTX, CTX_DIM), jnp.float32),
            pltpu.SemaphoreType.DMA,
            pltpu.SemaphoreType.DMA,
            pltpu.SemaphoreType.DMA,
            pltpu.SemaphoreType.DMA,
            pltpu.SemaphoreType.DMA,
            pltpu.SemaphoreType.DMA,
            pltpu.SemaphoreType.DMA,
            pltpu.SemaphoreType.DMA,
            pltpu.SemaphoreType.DMA,
        ],
    )
    return run(prefixs, ctx, suffixs)


# speedup vs baseline: 1.3076x; 1.3076x over previous
import functools

import jax
import jax.numpy as jnp
from jax.experimental import pallas as pl
from jax.experimental.pallas import tpu as pltpu

N_CLS = 1000
N_CTX = 4
CTX_DIM = 512
CTX_LEN = 77
SUFFIX_LEN = CTX_LEN - 1 - N_CTX

CB = 100


def _body(prefix_ref, ctx_ref, suffix_ref, out_ref):
    out_ref[:, 0:1, :] = prefix_ref[...]
    out_ref[:, 1:1 + N_CTX, :] = jnp.broadcast_to(
        ctx_ref[...][None], (CB, N_CTX, CTX_DIM))
    out_ref[:, 1 + N_CTX:, :] = suffix_ref[...]


def kernel(prefixs, ctx, suffixs):
    grid = (N_CLS // CB,)
    return pl.pallas_call(
        _body,
        grid=grid,
        in_specs=[
            pl.BlockSpec((CB, 1, CTX_DIM), lambda i: (i, 0, 0)),
            pl.BlockSpec((N_CTX, CTX_DIM), lambda i: (0, 0)),
            pl.BlockSpec((CB, SUFFIX_LEN, CTX_DIM), lambda i: (i, 0, 0)),
        ],
        out_specs=pl.BlockSpec((CB, CTX_LEN, CTX_DIM), lambda i: (i, 0, 0)),
        out_shape=jax.ShapeDtypeStruct((N_CLS, CTX_LEN, CTX_DIM), jnp.float32),
        compiler_params=pltpu.CompilerParams(
            dimension_semantics=("arbitrary",),
            vmem_limit_bytes=62 * 1024 * 1024,
        ),
    )(prefixs, ctx, suffixs)
